# baseline (speedup 1.0000x reference)
import jax
import jax.numpy as jnp
from jax.experimental import pallas as pl

_EPS = 1e-5
_ROWS = 2048


def _ln_body(x_ref, pos_ref, w_ref, b_ref, o_ref):
    x = x_ref[0] + pos_ref[...]
    mean = jnp.mean(x, axis=-1, keepdims=True)
    xc = x - mean
    var = jnp.mean(xc * xc, axis=-1, keepdims=True)
    normed = xc * jax.lax.rsqrt(var + _EPS)
    o_ref[0] = normed * w_ref[...] + b_ref[...]


@jax.jit
def kernel(input_embs, pos_table, ln_weight, ln_bias):
    B, S, H = input_embs.shape
    rows = _ROWS if S % _ROWS == 0 else S
    grid = (S // rows, B)
    return pl.pallas_call(
        _ln_body,
        grid=grid,
        in_specs=[
            pl.BlockSpec((1, rows, H), lambda i, b: (b, i, 0)),
            pl.BlockSpec((rows, H), lambda i, b: (i, 0)),
            pl.BlockSpec((1, H), lambda i, b: (0, 0)),
            pl.BlockSpec((1, H), lambda i, b: (0, 0)),
        ],
        out_specs=pl.BlockSpec((1, rows, H), lambda i, b: (b, i, 0)),
        out_shape=jax.ShapeDtypeStruct((B, S, H), input_embs.dtype),
    )(
        input_embs,
        pos_table[:S],
        ln_weight.reshape(1, H),
        ln_bias.reshape(1, H),
    )


# speedup vs baseline: 1.0297x; 1.0297x over previous
import jax
import jax.numpy as jnp
from jax.experimental import pallas as pl

_EPS = 1e-5
_ROWS = 2048


def _ln_body(x_ref, pos_ref, w_ref, b_ref, o_ref):
    x = x_ref[...] + pos_ref[...][None]
    mean = jnp.mean(x, axis=-1, keepdims=True)
    xc = x - mean
    var = jnp.mean(xc * xc, axis=-1, keepdims=True)
    normed = xc * jax.lax.rsqrt(var + _EPS)
    o_ref[...] = normed * w_ref[...] + b_ref[...]


@jax.jit
def kernel(input_embs, pos_table, ln_weight, ln_bias):
    B, S, H = input_embs.shape
    rows = 1024
    bpack = 2
    grid = (S // rows, B // bpack)
    return pl.pallas_call(
        _ln_body,
        grid=grid,
        in_specs=[
            pl.BlockSpec((bpack, rows, H), lambda i, b: (b, i, 0)),
            pl.BlockSpec((rows, H), lambda i, b: (i, 0)),
            pl.BlockSpec((1, H), lambda i, b: (0, 0)),
            pl.BlockSpec((1, H), lambda i, b: (0, 0)),
        ],
        out_specs=pl.BlockSpec((bpack, rows, H), lambda i, b: (b, i, 0)),
        out_shape=jax.ShapeDtypeStruct((B, S, H), input_embs.dtype),
    )(
        input_embs,
        pos_table[:S],
        ln_weight.reshape(1, H),
        ln_bias.reshape(1, H),
    )


# speedup vs baseline: 1.0501x; 1.0198x over previous
import jax
import jax.numpy as jnp
from jax.experimental import pallas as pl

_EPS = 1e-5
_ROWS = 2048


def _ln_body(x_ref, pos_ref, w_ref, b_ref, o_ref):
    x = x_ref[...] + pos_ref[...][None]
    mean = jnp.mean(x, axis=-1, keepdims=True)
    xc = x - mean
    var = jnp.mean(xc * xc, axis=-1, keepdims=True)
    normed = xc * jax.lax.rsqrt(var + _EPS)
    o_ref[...] = normed * w_ref[...] + b_ref[...]


@jax.jit
def kernel(input_embs, pos_table, ln_weight, ln_bias):
    B, S, H = input_embs.shape
    rows = 512
    bpack = 4
    grid = (S // rows, B // bpack)
    return pl.pallas_call(
        _ln_body,
        grid=grid,
        in_specs=[
            pl.BlockSpec((bpack, rows, H), lambda i, b: (b, i, 0)),
            pl.BlockSpec((rows, H), lambda i, b: (i, 0)),
            pl.BlockSpec((1, H), lambda i, b: (0, 0)),
            pl.BlockSpec((1, H), lambda i, b: (0, 0)),
        ],
        out_specs=pl.BlockSpec((bpack, rows, H), lambda i, b: (b, i, 0)),
        out_shape=jax.ShapeDtypeStruct((B, S, H), input_embs.dtype),
    )(
        input_embs,
        pos_table[:S],
        ln_weight.reshape(1, H),
        ln_bias.reshape(1, H),
    )
